# baseline (speedup 1.0000x reference)
import functools

import jax
import jax.numpy as jnp
from jax import lax
from jax.experimental import pallas as pl
from jax.experimental.pallas import tpu as pltpu
from jax.experimental.pallas import tpu_sc as plsc

_NE = 1000000
_DIM = 64
_TOT = 16384 * 26
_NW = 32
_PER_W = _TOT // _NW
_IR = 128
_NIR = _PER_W // _IR
_CH = 256
_NCH = _PER_W // _CH
_NSLOT = 4

_NSLAB = _NE // _IR
_TAIL = _NE - _NSLAB * _IR
_TPW = 246
_TSLOT = 3

_mesh = plsc.VectorSubcoreMesh(core_axis_name="c", subcore_axis_name="s")


def _make_transpose():
    @functools.partial(
        pl.kernel,
        mesh=_mesh,
        out_type=jax.ShapeDtypeStruct((_NE // 2, 2 * _DIM), jnp.float32),
        compiler_params=pltpu.CompilerParams(needs_layout_passes=False),
        scratch_types=[
            [pltpu.VMEM((_DIM, _IR + 8), jnp.float32)] * _TSLOT,
            [pltpu.VMEM((_DIM, _IR), jnp.float32)] * _TSLOT,
            pltpu.VMEM((_DIM, _DIM), jnp.float32),
            pltpu.VMEM((_DIM // 2, _IR), jnp.float32),
            [pltpu.SemaphoreType.DMA] * _TSLOT,
            [pltpu.SemaphoreType.DMA] * _TSLOT,
        ],
    )
    def transpose_kernel(wt_hbm, out_hbm, slab, tout, nslab, ntout, isem, osem):
        wid = lax.axis_index("s") * 2 + lax.axis_index("c")
        iota = lax.iota(jnp.int32, 16)
        rowv = [iota + 16 * m for m in range(4)]

        def slab_id(t):
            return lax.rem(wid + _NW * t, _NSLAB)

        def ifire(t, b):
            j = slab_id(t)
            pltpu.async_copy(
                wt_hbm.at[:, pl.ds(j * _IR, _IR)],
                slab[b].at[:, pl.ds(0, _IR)],
                isem[b],
            )

        def iwait(b):
            pltpu.make_async_copy(
                wt_hbm.at[:, pl.ds(0, _IR)],
                slab[b].at[:, pl.ds(0, _IR)],
                isem[b],
            ).wait()

        def ofire(t, b):
            j = slab_id(t)
            pltpu.async_copy(
                tout[b], out_hbm.at[pl.ds(j * _DIM, _DIM)], osem[b]
            )

        def owait(b):
            pltpu.make_async_copy(
                tout[b], out_hbm.at[pl.ds(0, _DIM)], osem[b]
            ).wait()

        def compute(b):
            @plsc.parallel_loop(0, _DIM, unroll=8)
            def _(a):
                for bc in range(8):
                    col = jnp.full((16,), bc // 4, jnp.int32) + 2 * a
                    v = plsc.load_gather(slab[b], [rowv[bc % 4], col])
                    tout[b][a, pl.ds(bc * 16, 16)] = v

        for b in range(_TSLOT):
            ifire(b, b)

        for t in range(_TSLOT):
            iwait(t)
            compute(t)
            ofire(t, t)
            ifire(t + _TSLOT, t)

        def outer(o, carry):
            for b in range(_TSLOT):
                t = _TSLOT + o * _TSLOT + b
                iwait(b)
                owait(b)
                compute(b)
                ofire(t, b)
                ifire(t + _TSLOT, b)
            return carry

        lax.fori_loop(0, (_TPW - 2 * _TSLOT) // _TSLOT, outer, 0)

        for e in range(_TSLOT):
            t = _TPW - _TSLOT + e
            b = t % _TSLOT
            iwait(b)
            owait(b)
            compute(b)
            ofire(t, b)
        for b in range(_TSLOT):
            owait(b)

        @pl.when(wid == 0)
        def _():
            pltpu.sync_copy(wt_hbm.at[:, pl.ds(_NSLAB * _IR, _TAIL)], nslab)

            @plsc.parallel_loop(0, _DIM // 2, unroll=8)
            def _nbody(a):
                for bc in range(8):
                    col = jnp.full((16,), bc // 4, jnp.int32) + 2 * a
                    v = plsc.load_gather(nslab, [rowv[bc % 4], col])
                    ntout[a, pl.ds(bc * 16, 16)] = v
            pltpu.sync_copy(
                ntout, out_hbm.at[pl.ds(_NSLAB * _DIM, _DIM // 2)]
            )

    return transpose_kernel


def _make_gather():
    @functools.partial(
        pl.kernel,
        mesh=_mesh,
        out_type=jax.ShapeDtypeStruct((_TOT, _DIM), jnp.float32),
        compiler_params=pltpu.CompilerParams(use_tc_tiling_on_sc=False),
        scratch_types=[
            pltpu.VMEM((_NIR, _IR), jnp.int32),
            [pltpu.VMEM((_CH, _DIM), jnp.float32)] * _NSLOT,
            [pltpu.SemaphoreType.DMA] * _NSLOT,
            [pltpu.SemaphoreType.DMA] * _NSLOT,
        ],
    )
    def gather_kernel(idx_hbm, table_hbm, out_hbm, idx_v, rows, gsem, ssem):
        wid = lax.axis_index("s") * 2 + lax.axis_index("c")
        pltpu.sync_copy(idx_hbm.at[pl.ds(wid * _NIR, _NIR)], idx_v)
        base = wid * _PER_W

        def gfire(ci, s):
            for k in range(_CH // _IR):
                pltpu.async_copy(
                    table_hbm.at[idx_v.at[ci * (_CH // _IR) + k]],
                    rows[s].at[pl.ds(k * _IR, _IR)],
                    gsem[s],
                )

        def gwait(s):
            pltpu.make_async_copy(
                out_hbm.at[pl.ds(0, _CH)], rows[s], gsem[s]
            ).wait()

        def sfire(ci, s):
            pltpu.async_copy(
                rows[s], out_hbm.at[pl.ds(base + ci * _CH, _CH)], ssem[s]
            )

        def swait(s):
            pltpu.make_async_copy(
                rows[s], out_hbm.at[pl.ds(base, _CH)], ssem[s]
            ).wait()

        for s in range(_NSLOT):
            gfire(s, s)
        gwait(0)
        sfire(0, 0)

        def outer(j, carry):
            for b in range(_NSLOT):
                i = _NSLOT + j * _NSLOT + b
                swait(b)
                gfire(i, b)
                b2 = (b + 1) % _NSLOT
                gwait(b2)
                sfire(i - (_NSLOT - 1), b2)
            return carry

        lax.fori_loop(0, (_NCH - _NSLOT) // _NSLOT, outer, 0)

        for e in range(_NSLOT - 1):
            i = _NCH + e
            b2 = (i + 1) % _NSLOT
            gwait(b2)
            sfire(i - (_NSLOT - 1), b2)
        for s in range(_NSLOT):
            swait(s)

    return gather_kernel


_TRANSPOSE = _make_transpose()
_GATHER = _make_gather()


def kernel(indices, weight):
    flat = indices.reshape(_TOT // _IR, _IR)
    table2 = _TRANSPOSE(weight.T)
    table = table2.reshape(_NE, _DIM)
    out = _GATHER(flat, table)
    return out.reshape(indices.shape + (weight.shape[-1],))


# speedup vs baseline: 1.2104x; 1.2104x over previous
import functools

import jax
import jax.numpy as jnp
from jax import lax
from jax.experimental import pallas as pl
from jax.experimental.pallas import tpu as pltpu
from jax.experimental.pallas import tpu_sc as plsc

_DIM = 64
_TOT = 16384 * 26
_NW = 32
_PER_W = _TOT // _NW
_IR = 128
_NIR = _PER_W // _IR
_CH = 256
_NCH = _PER_W // _CH
_NSLOT = 4

_mesh = plsc.VectorSubcoreMesh(core_axis_name="c", subcore_axis_name="s")


def _make_gather():
    @functools.partial(
        pl.kernel,
        mesh=_mesh,
        out_type=jax.ShapeDtypeStruct((_TOT, _DIM), jnp.float32),
        compiler_params=pltpu.CompilerParams(use_tc_tiling_on_sc=False),
        scratch_types=[
            pltpu.VMEM((_NIR, _IR), jnp.int32),
            [pltpu.VMEM((_CH, _DIM), jnp.float32)] * _NSLOT,
            [pltpu.SemaphoreType.DMA] * _NSLOT,
            [pltpu.SemaphoreType.DMA] * _NSLOT,
        ],
    )
    def gather_kernel(idx_hbm, table_hbm, out_hbm, idx_v, rows, gsem, ssem):
        wid = lax.axis_index("s") * 2 + lax.axis_index("c")
        pltpu.sync_copy(idx_hbm.at[pl.ds(wid * _NIR, _NIR)], idx_v)
        base = wid * _PER_W

        def gfire(ci, s):
            for k in range(_CH // _IR):
                pltpu.async_copy(
                    table_hbm.at[idx_v.at[ci * (_CH // _IR) + k]],
                    rows[s].at[pl.ds(k * _IR, _IR)],
                    gsem[s],
                )

        def gwait(s):
            pltpu.make_async_copy(
                out_hbm.at[pl.ds(0, _CH)], rows[s], gsem[s]
            ).wait()

        def sfire(ci, s):
            pltpu.async_copy(
                rows[s], out_hbm.at[pl.ds(base + ci * _CH, _CH)], ssem[s]
            )

        def swait(s):
            pltpu.make_async_copy(
                rows[s], out_hbm.at[pl.ds(base, _CH)], ssem[s]
            ).wait()

        for s in range(_NSLOT):
            gfire(s, s)
        gwait(0)
        sfire(0, 0)

        def outer(j, carry):
            for b in range(_NSLOT):
                i = _NSLOT + j * _NSLOT + b
                swait(b)
                gfire(i, b)
                b2 = (b + 1) % _NSLOT
                gwait(b2)
                sfire(i - (_NSLOT - 1), b2)
            return carry

        lax.fori_loop(0, (_NCH - _NSLOT) // _NSLOT, outer, 0)

        for e in range(_NSLOT - 1):
            i = _NCH + e
            b2 = (i + 1) % _NSLOT
            gwait(b2)
            sfire(i - (_NSLOT - 1), b2)
        for s in range(_NSLOT):
            swait(s)

    return gather_kernel


_GATHER = _make_gather()


def kernel(indices, weight):
    flat = indices.reshape(_TOT // _IR, _IR)
    out = _GATHER(flat, weight)
    return out.reshape(indices.shape + (weight.shape[-1],))
